# baseline (speedup 1.0000x reference)
import functools

import jax
import jax.numpy as jnp
from jax.experimental import pallas as pl
from jax.experimental.pallas import tpu as pltpu

B, N, T, P, D, O = 16, 2048, 1, 6, 128, 128
ROWS = B * N
BLK = 512


def _tc_body(embs_ref, vis_ref, w_ref, b_ref, mask_ref, out_ref):
    e = embs_ref[...]
    v = vis_ref[...]
    combined = e[:, 0, :]
    for p in range(1, P):
        combined += v[:, p][:, None] * e[:, p, :]
    s = 1.0 + jnp.sum(v[:, 1:], axis=1, keepdims=True)
    y = jnp.dot(combined, w_ref[...], preferred_element_type=jnp.float32)
    y = y + s * b_ref[...]
    out_ref[...] = jnp.where(mask_ref[...] > 0, y, 0.0)


@jax.jit
def kernel(embs, vis, W, b, masks):
    embs2 = embs.reshape(ROWS, P, D)
    vis2 = vis.reshape(ROWS, P)
    mask2 = masks.reshape(ROWS, 1).astype(jnp.float32)
    wt = W.T
    b2 = b.reshape(1, O)
    grid = (ROWS // BLK,)
    out = pl.pallas_call(
        _tc_body,
        grid=grid,
        in_specs=[
            pl.BlockSpec((BLK, P, D), lambda i: (i, 0, 0)),
            pl.BlockSpec((BLK, P), lambda i: (i, 0)),
            pl.BlockSpec((D, O), lambda i: (0, 0)),
            pl.BlockSpec((1, O), lambda i: (0, 0)),
            pl.BlockSpec((BLK, 1), lambda i: (i, 0)),
        ],
        out_specs=pl.BlockSpec((BLK, O), lambda i: (i, 0)),
        out_shape=jax.ShapeDtypeStruct((ROWS, O), jnp.float32),
    )(embs2, vis2, wt, b2, mask2)
    return out.reshape(B, N, O)
